# baseline (speedup 1.0000x reference)
import functools

import jax
import jax.numpy as jnp
from jax import lax
from jax.experimental import pallas as pl
from jax.experimental.pallas import tpu as pltpu
from jax.experimental.pallas import tpu_sc as plsc

_NB_CODE = 1024
_CODE_DIM = 512
_W = 512
_B = 8
_T = 64
_NPARTS = 6
_CIN = 128


def _values_term_k(i):
    i -= 1
    return ([4 + i * 3, 4 + i * 3 + 1, 4 + i * 3 + 2]
            + [4 + 63 + i * 6 + k for k in range(6)]
            + [4 + 63 + 126 + (i + 1) * 3 + k for k in range(3)])


def _part_indices():
    return [[0, 1, 2, 3, 4 + 63 + 126, 4 + 63 + 126 + 1, 4 + 63 + 126 + 2],
            [x for i in [3, 6, 9, 12, 15] for x in _values_term_k(i)],
            [x for i in [13, 16, 18, 20] for x in _values_term_k(i)],
            [x for i in [14, 17, 19, 21] for x in _values_term_k(i)],
            [x for i in [1, 4, 7, 10] for x in _values_term_k(i)] + [259, 260],
            [x for i in [2, 5, 8, 11] for x in _values_term_k(i)] + [261, 262]]


def _xform(w):
    o, i, k = w.shape
    return (w.transpose(2, 0, 1)
            .reshape(k, o // 8, 8, i // 128, 128)
            .transpose(0, 1, 3, 2, 4))


def _wmat(wbuf, slot, j, ni):
    wv = wbuf[slot, j]
    return jnp.concatenate(
        [wv[:, b].reshape(_W, 128) for b in range(ni)], axis=1)


def _conv3(h, wtap, bias, dil):
    b, t, c = h.shape
    z = jnp.zeros((b, dil, c), jnp.float32)
    xp = jnp.concatenate([z, h, z], axis=1)
    acc = None
    for j in range(3):
        xs = xp[:, j * dil:j * dil + t, :].reshape(b * t, c)
        pj = lax.dot_general(xs, wtap(j), (((1,), (1,)), ((), ())),
                             preferred_element_type=jnp.float32)
        acc = pj if acc is None else acc + pj
    acc = acc.reshape(b, t, acc.shape[-1])
    if bias is not None:
        acc = acc + bias[None, None, :]
    return acc


def _down4(h, wtap, bias):
    b, t, c = h.shape
    z = jnp.zeros((b, 1, c), jnp.float32)
    xp = jnp.concatenate([z, h, z], axis=1)
    to = t // 2
    ev = xp[:, :t, :].reshape(b, to, 2, c)
    od = xp[:, 2:, :].reshape(b, to, 2, c)
    taps = [ev[:, :, 0, :], ev[:, :, 1, :], od[:, :, 0, :], od[:, :, 1, :]]
    acc = None
    for j in range(4):
        xs = taps[j].reshape(b * to, c)
        pj = lax.dot_general(xs, wtap(j), (((1,), (1,)), ((), ())),
                             preferred_element_type=jnp.float32)
        acc = pj if acc is None else acc + pj
    return acc.reshape(b, to, c) + bias[None, None, :]


def _enc_sources(enc_params):
    srcs, steps = [], []

    def add(a, k, dbase, ni):
        srcs.append(a)
        return (len(srcs) - 1, k, dbase, ni)

    for p in enc_params:
        ci = jnp.pad(p["conv_in"]["w"],
                     ((0, 0), (0, _CIN - p["conv_in"]["w"].shape[1]), (0, 0)))
        steps.append([add(_xform(ci), 3, 0, 1)])
        for dblk in p["downs"]:
            steps.append([add(_xform(dblk["down"]["w"]), 4, 0, 4)])
            for rb in dblk["res"]:
                steps.append([add(_xform(rb["c1"]["w"]), 3, 0, 4),
                              add(_xform(rb["c2"]["w"]), 1, 3, 4)])
        steps.append([add(_xform(p["conv_out"]["w"]), 3, 0, 4)])
    return srcs, steps


def _make_enc_body(steps, nsrc):
    def body(*refs):
        wsrc = refs[:nsrc]
        x_ref, b_ref, out_ref, act_ref, wbuf, sem0, sem1 = refs[nsrc:]
        sems = [sem0, sem1]

        def copies(s):
            slot = s % 2
            out = []
            for (si, k, dbase, ni) in steps[s]:
                for j in range(k):
                    out.append(pltpu.make_async_copy(
                        wsrc[si].at[j],
                        wbuf.at[slot, dbase + j, :, :ni], sems[slot]))
            return out

        def res(p, l, t, dil):
            slot = (14 * p + l) % 2
            h = act_ref[:, :t, :]
            r = jnp.maximum(h, 0.0)
            r = _conv3(r, lambda j: _wmat(wbuf, slot, j, 4), b_ref[p, l, 0], dil)
            r = jnp.maximum(r, 0.0)
            r = (lax.dot_general(r.reshape(_B * t, _W), _wmat(wbuf, slot, 3, 4),
                                 (((1,), (1,)), ((), ())),
                                 preferred_element_type=jnp.float32)
                 + b_ref[p, l, 1][None, :]).reshape(_B, t, _W)
            act_ref[:, :t, :] = h + r

        def down(p, l, t):
            slot = (14 * p + l) % 2
            act_ref[:, :t // 2, :] = _down4(
                act_ref[:, :t, :], lambda j: _wmat(wbuf, slot, j, 4),
                b_ref[p, l, 0])

        def cin(p, l):
            slot = (14 * p + l) % 2
            h = _conv3(x_ref[p], lambda j: _wmat(wbuf, slot, j, 1),
                       b_ref[p, l, 0], 1)
            act_ref[:, :, :] = jnp.maximum(h, 0.0)

        def cout(p, l):
            slot = (14 * p + l) % 2
            f = _conv3(act_ref[:, :8, :], lambda j: _wmat(wbuf, slot, j, 4),
                       b_ref[p, l, 0], 1)
            s = jnp.sum(f * f, axis=(1, 2))
            out_ref[p] = f / jnp.sqrt(s)[:, None, None]

        for c in copies(0):
            c.start()
        for s in range(len(steps)):
            if s + 1 < len(steps):
                for c in copies(s + 1):
                    c.start()
            for c in copies(s):
                c.wait()
            p, l = divmod(s, 14)
            if l == 0:
                cin(p, l)
            elif l in (1, 5, 9):
                down(p, l, _T >> ((l - 1) // 4))
            elif l == 13:
                cout(p, l)
            else:
                blk = (l - 2) // 4
                res(p, l, _T >> (blk + 1), 3 ** (l - 2 - 4 * blk))

    return body


def _dec_sources(dec_params):
    srcs, steps = [], []

    def add(a, k, dbase, ni):
        srcs.append(a)
        return (len(srcs) - 1, k, dbase, ni)

    wci = _xform(dec_params["conv_in"]["w"])
    for c in range(6):
        steps.append([add(wci[:, :, 4 * c:4 * c + 4], 3, 0, 4)])
    for u in dec_params["ups"]:
        for rb in u["res"]:
            steps.append([add(_xform(rb["c1"]["w"]), 3, 0, 4),
                          add(_xform(rb["c2"]["w"]), 1, 3, 4)])
        steps.append([add(_xform(u["conv"]["w"]), 3, 0, 4)])
    steps.append([add(_xform(dec_params["conv_mid"]["w"]), 3, 0, 4)])
    co = jnp.pad(dec_params["conv_out"]["w"], ((0, _W - 263), (0, 0), (0, 0)))
    steps.append([add(_xform(co), 3, 0, 4)])
    return srcs, steps


def _make_dec_body(steps, nsrc):
    def body(*refs):
        wsrc = refs[:nsrc]
        zq_ref, b_ref, out_ref, act_ref, wbuf, sem0, sem1 = refs[nsrc:]
        sems = [sem0, sem1]

        def copies(s):
            slot = s % 2
            out = []
            for (si, k, dbase, ni) in steps[s]:
                for j in range(k):
                    out.append(pltpu.make_async_copy(
                        wsrc[si].at[j],
                        wbuf.at[slot, dbase + j, :, :ni], sems[slot]))
            return out

        def wtap(l):
            return lambda j: _wmat(wbuf, l % 2, j, 4)

        def cin(c):
            y = _conv3(zq_ref[c], wtap(c), None, 1)
            if c == 0:
                act_ref[:, :8, :] = y
            else:
                acc = act_ref[:, :8, :] + y
                if c == 5:
                    acc = jnp.maximum(acc + b_ref[c, 0][None, None, :], 0.0)
                act_ref[:, :8, :] = acc

        def res(l, t, dil):
            h = act_ref[:, :t, :]
            r = jnp.maximum(h, 0.0)
            r = _conv3(r, wtap(l), b_ref[l, 0], dil)
            r = jnp.maximum(r, 0.0)
            r = (lax.dot_general(r.reshape(_B * t, _W), _wmat(wbuf, l % 2, 3, 4),
                                 (((1,), (1,)), ((), ())),
                                 preferred_element_type=jnp.float32)
                 + b_ref[l, 1][None, :]).reshape(_B, t, _W)
            act_ref[:, :t, :] = h + r

        def up(l, t):
            h = act_ref[:, :t, :]
            hr = jnp.broadcast_to(h[:, :, None, :], (_B, t, 2, _W))
            hr = hr.reshape(_B, 2 * t, _W)
            act_ref[:, :2 * t, :] = _conv3(hr, wtap(l), b_ref[l, 0], 1)

        for c in copies(0):
            c.start()
        for s in range(len(steps)):
            if s + 1 < len(steps):
                for c in copies(s + 1):
                    c.start()
            for c in copies(s):
                c.wait()
            if s < 6:
                cin(s)
            elif s == 18:
                act_ref[:, :, :] = jnp.maximum(
                    _conv3(act_ref[:, :, :], wtap(s), b_ref[s, 0], 1), 0.0)
            elif s == 19:
                out_ref[:, :, :] = _conv3(act_ref[:, :, :], wtap(s),
                                          b_ref[s, 0], 1)
            else:
                blk = (s - 6) // 4
                r = (s - 6) % 4
                t = 8 << blk
                if r == 3:
                    up(s, t)
                else:
                    res(s, t, 3 ** (2 - r))

    return body


def _quant_body(feat_ref, emb_ref, idx_ref, loss_ref, perp_ref):
    z = feat_ref[0].reshape(_B * 8, _CODE_DIM)
    emb = emb_ref[0]
    prod = lax.dot_general(z, emb, (((1,), (1,)), ((), ())),
                           preferred_element_type=jnp.float32)
    d = (jnp.sum(z * z, axis=1, keepdims=True)
         + jnp.sum(emb * emb, axis=1)[None, :] - 2.0 * prod)
    idx = jnp.argmin(d, axis=1).astype(jnp.int32)
    onehot = (idx[:, None]
              == lax.broadcasted_iota(jnp.int32, (_B * 8, _NB_CODE), 1)
              ).astype(jnp.float32)
    zq = jnp.dot(onehot, emb, preferred_element_type=jnp.float32)
    loss = 2.0 * jnp.mean((zq - z) ** 2)
    e_mean = jnp.sum(onehot, axis=0) / float(_B * 8)
    perp = jnp.exp(-jnp.sum(e_mean * jnp.log(e_mean + 1e-10)))
    idx_ref[0, 0] = idx
    loss_ref[0, 0] = jnp.broadcast_to(loss, (128,))
    perp_ref[0, 0] = jnp.broadcast_to(perp, (128,))


def _sc_gather(emb_flat, gidx):
    info = plsc.get_sparse_core_info()
    nc, ns = info.num_cores, info.num_subcores
    bpw = 512 // (nc * ns)
    mesh = plsc.VectorSubcoreMesh(core_axis_name="c", subcore_axis_name="s")

    @functools.partial(
        pl.kernel, mesh=mesh,
        out_type=jax.ShapeDtypeStruct((512, _CODE_DIM), jnp.float32),
        scratch_types=[pltpu.VMEM((bpw,), jnp.int32),
                       pltpu.VMEM((bpw, _CODE_DIM), jnp.float32),
                       pltpu.SemaphoreType.DMA])
    def k(emb_hbm, idx_hbm, out_hbm, idx_v, rows_v, sem):
        wid = lax.axis_index("s") * nc + lax.axis_index("c")
        base = wid * bpw
        pltpu.sync_copy(idx_hbm.at[pl.ds(base, bpw)], idx_v)
        pltpu.async_copy(emb_hbm.at[idx_v], rows_v, sem).wait()
        pltpu.sync_copy(rows_v, out_hbm.at[pl.ds(base, bpw)])

    return k(emb_flat, gidx)


def _zb():
    return jnp.zeros((_W,), jnp.float32)


def kernel(x, enc_params, quant_emb, dec_params):
    parts = _part_indices()

    xps = []
    for idxs in parts:
        xc = jnp.transpose(jnp.take(x, jnp.array(idxs), axis=1), (0, 2, 1))
        xps.append(jnp.pad(xc, ((0, 0), (0, 0), (0, _CIN - len(idxs)))))
    x_parts = jnp.stack(xps)

    ebs = []
    for p in enc_params:
        bias2 = [jnp.stack([p["conv_in"]["b"], _zb()])]
        for dblk in p["downs"]:
            bias2.append(jnp.stack([dblk["down"]["b"], _zb()]))
            for rb in dblk["res"]:
                bias2.append(jnp.stack([rb["c1"]["b"], rb["c2"]["b"]]))
        bias2.append(jnp.stack([p["conv_out"]["b"], _zb()]))
        ebs.append(jnp.stack(bias2))
    enc_b = jnp.stack(ebs)

    dbs = []
    for c in range(6):
        dbs.append(jnp.stack(
            [dec_params["conv_in"]["b"] if c == 5 else _zb(), _zb()]))
    for u in dec_params["ups"]:
        for rb in u["res"]:
            dbs.append(jnp.stack([rb["c1"]["b"], rb["c2"]["b"]]))
        dbs.append(jnp.stack([u["conv"]["b"], _zb()]))
    dbs.append(jnp.stack([dec_params["conv_mid"]["b"], _zb()]))
    dbs.append(jnp.stack([jnp.pad(dec_params["conv_out"]["b"], (0, _W - 263)),
                          _zb()]))
    dec_b = jnp.stack(dbs)

    emb_stack = jnp.stack(quant_emb)

    esrcs, esteps = _enc_sources(enc_params)
    feat = pl.pallas_call(
        _make_enc_body(esteps, len(esrcs)),
        in_specs=[pl.BlockSpec(memory_space=pl.ANY)] * len(esrcs)
        + [pl.BlockSpec(memory_space=pltpu.VMEM)] * 2,
        out_shape=jax.ShapeDtypeStruct((_NPARTS, _B, 8, _W), jnp.float32),
        scratch_shapes=[pltpu.VMEM((_B, _T, _W), jnp.float32),
                        pltpu.VMEM((2, 4, 64, 4, 8, 128), jnp.float32),
                        pltpu.SemaphoreType.DMA, pltpu.SemaphoreType.DMA],
    )(*esrcs, x_parts, enc_b)

    return feat, x_parts.sum(), enc_b.sum()

    idx, loss_arr, perp_arr = pl.pallas_call(
        _quant_body,
        grid=(_NPARTS,),
        in_specs=[
            pl.BlockSpec((1, _B, 8, _W), lambda p: (p, 0, 0, 0)),
            pl.BlockSpec((1, _NB_CODE, _CODE_DIM), lambda p: (p, 0, 0)),
        ],
        out_specs=[
            pl.BlockSpec((1, 1, 64), lambda p: (p, 0, 0)),
            pl.BlockSpec((1, 1, 128), lambda p: (p, 0, 0)),
            pl.BlockSpec((1, 1, 128), lambda p: (p, 0, 0)),
        ],
        out_shape=[
            jax.ShapeDtypeStruct((_NPARTS, 1, 64), jnp.int32),
            jax.ShapeDtypeStruct((_NPARTS, 1, 128), jnp.float32),
            jax.ShapeDtypeStruct((_NPARTS, 1, 128), jnp.float32),
        ],
        compiler_params=pltpu.CompilerParams(
            dimension_semantics=("arbitrary",)),
    )(feat, emb_stack)

    gidx = (idx.reshape(_NPARTS, 64)
            + _NB_CODE * jnp.arange(_NPARTS, dtype=jnp.int32)[:, None]
            ).reshape(-1)
    gidx = jnp.concatenate([gidx, jnp.zeros((128,), jnp.int32)])
    zq_rows = _sc_gather(emb_stack.reshape(-1, _CODE_DIM), gidx)
    zq = zq_rows[:_NPARTS * 64].reshape(_NPARTS, _B, 8, _CODE_DIM)

    dsrcs, dsteps = _dec_sources(dec_params)
    dec_out = pl.pallas_call(
        _make_dec_body(dsteps, len(dsrcs)),
        in_specs=[pl.BlockSpec(memory_space=pl.ANY)] * len(dsrcs)
        + [pl.BlockSpec(memory_space=pltpu.VMEM)] * 2,
        out_shape=jax.ShapeDtypeStruct((_B, _T, _W), jnp.float32),
        scratch_shapes=[pltpu.VMEM((_B, _T, _W), jnp.float32),
                        pltpu.VMEM((2, 4, 64, 4, 8, 128), jnp.float32),
                        pltpu.SemaphoreType.DMA, pltpu.SemaphoreType.DMA],
    )(*dsrcs, zq, dec_b)

    dec = jnp.transpose(dec_out[:, :, :263], (0, 2, 1))[:, :, None, :]
    loss = jnp.sum(loss_arr[:, 0, 0])
    perplexity = jnp.sum(perp_arr[:, 0, 0])
    return dec, loss, perplexity


# speedup vs baseline: 1.4197x; 1.4197x over previous
import functools

import jax
import jax.numpy as jnp
from jax import lax
from jax.experimental import pallas as pl
from jax.experimental.pallas import tpu as pltpu
from jax.experimental.pallas import tpu_sc as plsc

_NB_CODE = 1024
_CODE_DIM = 512
_W = 512
_B = 8
_T = 64
_NPARTS = 6
_CIN = 128


def _values_term_k(i):
    i -= 1
    return ([4 + i * 3, 4 + i * 3 + 1, 4 + i * 3 + 2]
            + [4 + 63 + i * 6 + k for k in range(6)]
            + [4 + 63 + 126 + (i + 1) * 3 + k for k in range(3)])


def _part_indices():
    return [[0, 1, 2, 3, 4 + 63 + 126, 4 + 63 + 126 + 1, 4 + 63 + 126 + 2],
            [x for i in [3, 6, 9, 12, 15] for x in _values_term_k(i)],
            [x for i in [13, 16, 18, 20] for x in _values_term_k(i)],
            [x for i in [14, 17, 19, 21] for x in _values_term_k(i)],
            [x for i in [1, 4, 7, 10] for x in _values_term_k(i)] + [259, 260],
            [x for i in [2, 5, 8, 11] for x in _values_term_k(i)] + [261, 262]]


def _xform(w):
    o, i, k = w.shape
    return (w.transpose(2, 0, 1)
            .reshape(k, o // 8, 8, i // 128, 128)
            .transpose(0, 1, 3, 2, 4))


def _wmat(wbuf, slot, j, ni):
    wv = wbuf[slot, j]
    return jnp.concatenate(
        [wv[:, b].reshape(_W, 128) for b in range(ni)], axis=1)


def _conv3(h, wtap, bias, dil):
    b, t, c = h.shape
    z = jnp.zeros((b, dil, c), jnp.float32)
    xp = jnp.concatenate([z, h, z], axis=1)
    acc = None
    for j in range(3):
        xs = xp[:, j * dil:j * dil + t, :].reshape(b * t, c)
        pj = lax.dot_general(xs, wtap(j), (((1,), (1,)), ((), ())),
                             preferred_element_type=jnp.float32)
        acc = pj if acc is None else acc + pj
    acc = acc.reshape(b, t, acc.shape[-1])
    if bias is not None:
        acc = acc + bias[None, None, :]
    return acc


def _down4(h, wtap, bias):
    b, t, c = h.shape
    z = jnp.zeros((b, 1, c), jnp.float32)
    xp = jnp.concatenate([z, h, z], axis=1)
    to = t // 2
    ev = xp[:, :t, :].reshape(b, to, 2, c)
    od = xp[:, 2:, :].reshape(b, to, 2, c)
    taps = [ev[:, :, 0, :], ev[:, :, 1, :], od[:, :, 0, :], od[:, :, 1, :]]
    acc = None
    for j in range(4):
        xs = taps[j].reshape(b * to, c)
        pj = lax.dot_general(xs, wtap(j), (((1,), (1,)), ((), ())),
                             preferred_element_type=jnp.float32)
        acc = pj if acc is None else acc + pj
    return acc.reshape(b, to, c) + bias[None, None, :]


def _enc_sources(enc_params):
    srcs, steps = [], []

    def add(a, k, dbase, ni):
        srcs.append(a)
        return (len(srcs) - 1, k, dbase, ni)

    for p in enc_params:
        ci = jnp.pad(p["conv_in"]["w"],
                     ((0, 0), (0, _CIN - p["conv_in"]["w"].shape[1]), (0, 0)))
        steps.append([add(_xform(ci), 3, 0, 1)])
        for dblk in p["downs"]:
            steps.append([add(_xform(dblk["down"]["w"]), 4, 0, 4)])
            for rb in dblk["res"]:
                steps.append([add(_xform(rb["c1"]["w"]), 3, 0, 4),
                              add(_xform(rb["c2"]["w"]), 1, 3, 4)])
        steps.append([add(_xform(p["conv_out"]["w"]), 3, 0, 4)])
    return srcs, steps


def _make_enc_body(steps, nsrc):
    def body(*refs):
        wsrc = refs[:nsrc]
        x_ref, b_ref, out_ref, act_ref, wbuf, sem0, sem1 = refs[nsrc:]
        sems = [sem0, sem1]

        def copies(s):
            slot = s % 2
            out = []
            for (si, k, dbase, ni) in steps[s]:
                for j in range(k):
                    out.append(pltpu.make_async_copy(
                        wsrc[si].at[j],
                        wbuf.at[slot, dbase + j, :, :ni], sems[slot]))
            return out

        def res(p, l, t, dil):
            slot = (14 * p + l) % 2
            h = act_ref[:, :t, :]
            r = jnp.maximum(h, 0.0)
            r = _conv3(r, lambda j: _wmat(wbuf, slot, j, 4), b_ref[p, l, 0], dil)
            r = jnp.maximum(r, 0.0)
            r = (lax.dot_general(r.reshape(_B * t, _W), _wmat(wbuf, slot, 3, 4),
                                 (((1,), (1,)), ((), ())),
                                 preferred_element_type=jnp.float32)
                 + b_ref[p, l, 1][None, :]).reshape(_B, t, _W)
            act_ref[:, :t, :] = h + r

        def down(p, l, t):
            slot = (14 * p + l) % 2
            act_ref[:, :t // 2, :] = _down4(
                act_ref[:, :t, :], lambda j: _wmat(wbuf, slot, j, 4),
                b_ref[p, l, 0])

        def cin(p, l):
            slot = (14 * p + l) % 2
            h = _conv3(x_ref[p], lambda j: _wmat(wbuf, slot, j, 1),
                       b_ref[p, l, 0], 1)
            act_ref[:, :, :] = jnp.maximum(h, 0.0)

        def cout(p, l):
            slot = (14 * p + l) % 2
            f = _conv3(act_ref[:, :8, :], lambda j: _wmat(wbuf, slot, j, 4),
                       b_ref[p, l, 0], 1)
            s = jnp.sum(f * f, axis=(1, 2))
            out_ref[p] = f / jnp.sqrt(s)[:, None, None]

        for c in copies(0):
            c.start()
        for s in range(len(steps)):
            if s + 1 < len(steps):
                for c in copies(s + 1):
                    c.start()
            for c in copies(s):
                c.wait()
            p, l = divmod(s, 14)
            if l == 0:
                cin(p, l)
            elif l in (1, 5, 9):
                down(p, l, _T >> ((l - 1) // 4))
            elif l == 13:
                cout(p, l)
            else:
                blk = (l - 2) // 4
                res(p, l, _T >> (blk + 1), 3 ** (l - 2 - 4 * blk))

    return body


def _dec_sources(dec_params):
    srcs, steps = [], []

    def add(a, k, dbase, ni):
        srcs.append(a)
        return (len(srcs) - 1, k, dbase, ni)

    wci = _xform(dec_params["conv_in"]["w"])
    for c in range(6):
        steps.append([add(wci[:, :, 4 * c:4 * c + 4], 3, 0, 4)])
    for u in dec_params["ups"]:
        for rb in u["res"]:
            steps.append([add(_xform(rb["c1"]["w"]), 3, 0, 4),
                          add(_xform(rb["c2"]["w"]), 1, 3, 4)])
        steps.append([add(_xform(u["conv"]["w"]), 3, 0, 4)])
    steps.append([add(_xform(dec_params["conv_mid"]["w"]), 3, 0, 4)])
    co = jnp.pad(dec_params["conv_out"]["w"], ((0, _W - 263), (0, 0), (0, 0)))
    steps.append([add(_xform(co), 3, 0, 4)])
    return srcs, steps


def _make_dec_body(steps, nsrc):
    def body(*refs):
        wsrc = refs[:nsrc]
        zq_ref, b_ref, out_ref, act_ref, wbuf, sem0, sem1 = refs[nsrc:]
        sems = [sem0, sem1]

        def copies(s):
            slot = s % 2
            out = []
            for (si, k, dbase, ni) in steps[s]:
                for j in range(k):
                    out.append(pltpu.make_async_copy(
                        wsrc[si].at[j],
                        wbuf.at[slot, dbase + j, :, :ni], sems[slot]))
            return out

        def wtap(l):
            return lambda j: _wmat(wbuf, l % 2, j, 4)

        def cin(c):
            y = _conv3(zq_ref[c], wtap(c), None, 1)
            if c == 0:
                act_ref[:, :8, :] = y
            else:
                acc = act_ref[:, :8, :] + y
                if c == 5:
                    acc = jnp.maximum(acc + b_ref[c, 0][None, None, :], 0.0)
                act_ref[:, :8, :] = acc

        def res(l, t, dil):
            h = act_ref[:, :t, :]
            r = jnp.maximum(h, 0.0)
            r = _conv3(r, wtap(l), b_ref[l, 0], dil)
            r = jnp.maximum(r, 0.0)
            r = (lax.dot_general(r.reshape(_B * t, _W), _wmat(wbuf, l % 2, 3, 4),
                                 (((1,), (1,)), ((), ())),
                                 preferred_element_type=jnp.float32)
                 + b_ref[l, 1][None, :]).reshape(_B, t, _W)
            act_ref[:, :t, :] = h + r

        def up(l, t):
            h = act_ref[:, :t, :]
            hr = jnp.broadcast_to(h[:, :, None, :], (_B, t, 2, _W))
            hr = hr.reshape(_B, 2 * t, _W)
            act_ref[:, :2 * t, :] = _conv3(hr, wtap(l), b_ref[l, 0], 1)

        for c in copies(0):
            c.start()
        for s in range(len(steps)):
            if s + 1 < len(steps):
                for c in copies(s + 1):
                    c.start()
            for c in copies(s):
                c.wait()
            if s < 6:
                cin(s)
            elif s == 18:
                act_ref[:, :, :] = jnp.maximum(
                    _conv3(act_ref[:, :, :], wtap(s), b_ref[s, 0], 1), 0.0)
            elif s == 19:
                out_ref[:, :, :] = _conv3(act_ref[:, :, :], wtap(s),
                                          b_ref[s, 0], 1)
            else:
                blk = (s - 6) // 4
                r = (s - 6) % 4
                t = 8 << blk
                if r == 3:
                    up(s, t)
                else:
                    res(s, t, 3 ** (2 - r))

    return body


def _quant_body(feat_ref, emb_ref, idx_ref, loss_ref, perp_ref):
    z = feat_ref[0].reshape(_B * 8, _CODE_DIM)
    emb = emb_ref[0]
    prod = lax.dot_general(z, emb, (((1,), (1,)), ((), ())),
                           preferred_element_type=jnp.float32)
    d = (jnp.sum(z * z, axis=1, keepdims=True)
         + jnp.sum(emb * emb, axis=1)[None, :] - 2.0 * prod)
    idx = jnp.argmin(d, axis=1).astype(jnp.int32)
    onehot = (idx[:, None]
              == lax.broadcasted_iota(jnp.int32, (_B * 8, _NB_CODE), 1)
              ).astype(jnp.float32)
    zq = jnp.dot(onehot, emb, preferred_element_type=jnp.float32)
    loss = 2.0 * jnp.mean((zq - z) ** 2)
    e_mean = jnp.sum(onehot, axis=0) / float(_B * 8)
    perp = jnp.exp(-jnp.sum(e_mean * jnp.log(e_mean + 1e-10)))
    idx_ref[0, 0] = idx
    loss_ref[0, 0] = jnp.broadcast_to(loss, (128,))
    perp_ref[0, 0] = jnp.broadcast_to(perp, (128,))


def _sc_gather(emb_flat, gidx):
    info = plsc.get_sparse_core_info()
    nc, ns = info.num_cores, info.num_subcores
    bpw = 512 // (nc * ns)
    mesh = plsc.VectorSubcoreMesh(core_axis_name="c", subcore_axis_name="s")

    @functools.partial(
        pl.kernel, mesh=mesh,
        out_type=jax.ShapeDtypeStruct((512, _CODE_DIM), jnp.float32),
        scratch_types=[pltpu.VMEM((bpw,), jnp.int32),
                       pltpu.VMEM((bpw, _CODE_DIM), jnp.float32),
                       pltpu.SemaphoreType.DMA])
    def k(emb_hbm, idx_hbm, out_hbm, idx_v, rows_v, sem):
        wid = lax.axis_index("s") * nc + lax.axis_index("c")
        base = wid * bpw
        pltpu.sync_copy(idx_hbm.at[pl.ds(base, bpw)], idx_v)
        pltpu.async_copy(emb_hbm.at[idx_v], rows_v, sem).wait()
        pltpu.sync_copy(rows_v, out_hbm.at[pl.ds(base, bpw)])

    return k(emb_flat, gidx)


def _zb():
    return jnp.zeros((_W,), jnp.float32)


def kernel(x, enc_params, quant_emb, dec_params):
    parts = _part_indices()

    xps = []
    for idxs in parts:
        xc = jnp.transpose(jnp.take(x, jnp.array(idxs), axis=1), (0, 2, 1))
        xps.append(jnp.pad(xc, ((0, 0), (0, 0), (0, _CIN - len(idxs)))))
    x_parts = jnp.stack(xps)

    ebs = []
    for p in enc_params:
        bias2 = [jnp.stack([p["conv_in"]["b"], _zb()])]
        for dblk in p["downs"]:
            bias2.append(jnp.stack([dblk["down"]["b"], _zb()]))
            for rb in dblk["res"]:
                bias2.append(jnp.stack([rb["c1"]["b"], rb["c2"]["b"]]))
        bias2.append(jnp.stack([p["conv_out"]["b"], _zb()]))
        ebs.append(jnp.stack(bias2))
    enc_b = jnp.stack(ebs)

    dbs = []
    for c in range(6):
        dbs.append(jnp.stack(
            [dec_params["conv_in"]["b"] if c == 5 else _zb(), _zb()]))
    for u in dec_params["ups"]:
        for rb in u["res"]:
            dbs.append(jnp.stack([rb["c1"]["b"], rb["c2"]["b"]]))
        dbs.append(jnp.stack([u["conv"]["b"], _zb()]))
    dbs.append(jnp.stack([dec_params["conv_mid"]["b"], _zb()]))
    dbs.append(jnp.stack([jnp.pad(dec_params["conv_out"]["b"], (0, _W - 263)),
                          _zb()]))
    dec_b = jnp.stack(dbs)

    emb_stack = jnp.stack(quant_emb)

    esrcs, esteps = _enc_sources(enc_params)
    feat = pl.pallas_call(
        _make_enc_body(esteps, len(esrcs)),
        in_specs=[pl.BlockSpec(memory_space=pl.ANY)] * len(esrcs)
        + [pl.BlockSpec(memory_space=pltpu.VMEM)] * 2,
        out_shape=jax.ShapeDtypeStruct((_NPARTS, _B, 8, _W), jnp.float32),
        scratch_shapes=[pltpu.VMEM((_B, _T, _W), jnp.float32),
                        pltpu.VMEM((2, 4, 64, 4, 8, 128), jnp.float32),
                        pltpu.SemaphoreType.DMA, pltpu.SemaphoreType.DMA],
    )(*esrcs, x_parts, enc_b)

    idx, loss_arr, perp_arr = pl.pallas_call(
        _quant_body,
        grid=(_NPARTS,),
        in_specs=[
            pl.BlockSpec((1, _B, 8, _W), lambda p: (p, 0, 0, 0)),
            pl.BlockSpec((1, _NB_CODE, _CODE_DIM), lambda p: (p, 0, 0)),
        ],
        out_specs=[
            pl.BlockSpec((1, 1, 64), lambda p: (p, 0, 0)),
            pl.BlockSpec((1, 1, 128), lambda p: (p, 0, 0)),
            pl.BlockSpec((1, 1, 128), lambda p: (p, 0, 0)),
        ],
        out_shape=[
            jax.ShapeDtypeStruct((_NPARTS, 1, 64), jnp.int32),
            jax.ShapeDtypeStruct((_NPARTS, 1, 128), jnp.float32),
            jax.ShapeDtypeStruct((_NPARTS, 1, 128), jnp.float32),
        ],
        compiler_params=pltpu.CompilerParams(
            dimension_semantics=("arbitrary",)),
    )(feat, emb_stack)

    gidx = (idx.reshape(_NPARTS, 64)
            + _NB_CODE * jnp.arange(_NPARTS, dtype=jnp.int32)[:, None]
            ).reshape(-1)
    gidx = jnp.concatenate([gidx, jnp.zeros((128,), jnp.int32)])
    zq_rows = _sc_gather(emb_stack.reshape(-1, _CODE_DIM), gidx)
    zq = zq_rows[:_NPARTS * 64].reshape(_NPARTS, _B, 8, _CODE_DIM)

    dsrcs, dsteps = _dec_sources(dec_params)
    dec_out = pl.pallas_call(
        _make_dec_body(dsteps, len(dsrcs)),
        in_specs=[pl.BlockSpec(memory_space=pl.ANY)] * len(dsrcs)
        + [pl.BlockSpec(memory_space=pltpu.VMEM)] * 2,
        out_shape=jax.ShapeDtypeStruct((_B, _T, _W), jnp.float32),
        scratch_shapes=[pltpu.VMEM((_B, _T, _W), jnp.float32),
                        pltpu.VMEM((2, 4, 64, 4, 8, 128), jnp.float32),
                        pltpu.SemaphoreType.DMA, pltpu.SemaphoreType.DMA],
    )(*dsrcs, zq, dec_b)

    dec = jnp.transpose(dec_out[:, :, :263], (0, 2, 1))[:, :, None, :]
    loss = jnp.sum(loss_arr[:, 0, 0])
    perplexity = jnp.sum(perp_arr[:, 0, 0])
    return dec, loss, perplexity


# speedup vs baseline: 1.5833x; 1.1153x over previous
import functools

import jax
import jax.numpy as jnp
from jax import lax
from jax.experimental import pallas as pl
from jax.experimental.pallas import tpu as pltpu
from jax.experimental.pallas import tpu_sc as plsc

_NB_CODE = 1024
_CODE_DIM = 512
_W = 512
_B = 8
_T = 64
_NPARTS = 6
_CIN = 128
_RING = 4


def _values_term_k(i):
    i -= 1
    return ([4 + i * 3, 4 + i * 3 + 1, 4 + i * 3 + 2]
            + [4 + 63 + i * 6 + k for k in range(6)]
            + [4 + 63 + 126 + (i + 1) * 3 + k for k in range(3)])


def _part_indices():
    return [[0, 1, 2, 3, 4 + 63 + 126, 4 + 63 + 126 + 1, 4 + 63 + 126 + 2],
            [x for i in [3, 6, 9, 12, 15] for x in _values_term_k(i)],
            [x for i in [13, 16, 18, 20] for x in _values_term_k(i)],
            [x for i in [14, 17, 19, 21] for x in _values_term_k(i)],
            [x for i in [1, 4, 7, 10] for x in _values_term_k(i)] + [259, 260],
            [x for i in [2, 5, 8, 11] for x in _values_term_k(i)] + [261, 262]]


def _xform(w):
    o, i, k = w.shape
    return (w.transpose(2, 0, 1)
            .reshape(k, o // 8, 8, i // 128, 128)
            .transpose(0, 1, 3, 2, 4))


def _wmat(wbuf, slot, j, ni):
    wv = wbuf[slot, j]
    return jnp.concatenate(
        [wv[:, b].reshape(_W, 128) for b in range(ni)], axis=1)


def _conv3(h, wtap, bias, dil):
    b, t, c = h.shape
    z = jnp.zeros((b, dil, c), jnp.float32)
    xp = jnp.concatenate([z, h, z], axis=1)
    acc = None
    for j in range(3):
        xs = xp[:, j * dil:j * dil + t, :].reshape(b * t, c)
        pj = lax.dot_general(xs, wtap(j), (((1,), (1,)), ((), ())),
                             preferred_element_type=jnp.float32)
        acc = pj if acc is None else acc + pj
    acc = acc.reshape(b, t, acc.shape[-1])
    if bias is not None:
        acc = acc + bias[None, None, :]
    return acc


def _down4(h, wtap, bias):
    b, t, c = h.shape
    z = jnp.zeros((b, 1, c), jnp.float32)
    xp = jnp.concatenate([z, h, z], axis=1)
    to = t // 2
    ev = xp[:, :t, :].reshape(b, to, 2, c)
    od = xp[:, 2:, :].reshape(b, to, 2, c)
    taps = [ev[:, :, 0, :], ev[:, :, 1, :], od[:, :, 0, :], od[:, :, 1, :]]
    acc = None
    for j in range(4):
        xs = taps[j].reshape(b * to, c)
        pj = lax.dot_general(xs, wtap(j), (((1,), (1,)), ((), ())),
                             preferred_element_type=jnp.float32)
        acc = pj if acc is None else acc + pj
    return acc.reshape(b, to, c) + bias[None, None, :]


def _enc_sources(enc_params):
    srcs, steps = [], []

    def add(a, k, dbase, ni):
        srcs.append(a)
        return (len(srcs) - 1, k, dbase, ni)

    for p in enc_params:
        ci = jnp.pad(p["conv_in"]["w"],
                     ((0, 0), (0, _CIN - p["conv_in"]["w"].shape[1]), (0, 0)))
        steps.append([add(_xform(ci), 3, 0, 1)])
        for dblk in p["downs"]:
            steps.append([add(_xform(dblk["down"]["w"]), 4, 0, 4)])
            for rb in dblk["res"]:
                steps.append([add(_xform(rb["c1"]["w"]), 3, 0, 4),
                              add(_xform(rb["c2"]["w"]), 1, 3, 4)])
        steps.append([add(_xform(p["conv_out"]["w"]), 3, 0, 4)])
    return srcs, steps


def _make_enc_body(steps, nsrc):
    def body(*refs):
        wsrc = refs[:nsrc]
        x_ref, b_ref, out_ref, act_ref, wbuf, *sems = refs[nsrc:]

        def copies(s):
            slot = s % _RING
            out = []
            for e in steps[s]:
                (si, k, dbase, ni), chunk = e[:4], (e[4] if len(e) > 4 else None)
                for j in range(k):
                    src = wsrc[si].at[j]
                    if chunk is not None:
                        src = wsrc[si].at[j, :, pl.ds(4 * chunk, 4)]
                    out.append(pltpu.make_async_copy(
                        src, wbuf.at[slot, dbase + j, :, :ni], sems[slot]))
            return out

        def res(p, l, t, dil):
            slot = (14 * p + l) % _RING
            h = act_ref[:, :t, :]
            r = jnp.maximum(h, 0.0)
            r = _conv3(r, lambda j: _wmat(wbuf, slot, j, 4), b_ref[p, l, 0], dil)
            r = jnp.maximum(r, 0.0)
            r = (lax.dot_general(r.reshape(_B * t, _W), _wmat(wbuf, slot, 3, 4),
                                 (((1,), (1,)), ((), ())),
                                 preferred_element_type=jnp.float32)
                 + b_ref[p, l, 1][None, :]).reshape(_B, t, _W)
            act_ref[:, :t, :] = h + r

        def down(p, l, t):
            slot = (14 * p + l) % _RING
            act_ref[:, :t // 2, :] = _down4(
                act_ref[:, :t, :], lambda j: _wmat(wbuf, slot, j, 4),
                b_ref[p, l, 0])

        def cin(p, l):
            slot = (14 * p + l) % _RING
            h = _conv3(x_ref[p], lambda j: _wmat(wbuf, slot, j, 1),
                       b_ref[p, l, 0], 1)
            act_ref[:, :, :] = jnp.maximum(h, 0.0)

        def cout(p, l):
            slot = (14 * p + l) % _RING
            f = _conv3(act_ref[:, :8, :], lambda j: _wmat(wbuf, slot, j, 4),
                       b_ref[p, l, 0], 1)
            s = jnp.sum(f * f, axis=(1, 2))
            out_ref[p] = f / jnp.sqrt(s)[:, None, None]

        for s0 in range(min(_RING - 1, len(steps))):
            for c in copies(s0):
                c.start()
        for s in range(len(steps)):
            if s + _RING - 1 < len(steps):
                for c in copies(s + _RING - 1):
                    c.start()
            for c in copies(s):
                c.wait()
            p, l = divmod(s, 14)
            if l == 0:
                cin(p, l)
            elif l in (1, 5, 9):
                down(p, l, _T >> ((l - 1) // 4))
            elif l == 13:
                cout(p, l)
            else:
                blk = (l - 2) // 4
                res(p, l, _T >> (blk + 1), 3 ** (l - 2 - 4 * blk))

    return body


def _dec_sources(dec_params):
    srcs, steps = [], []

    def add(a, k, dbase, ni):
        srcs.append(a)
        return (len(srcs) - 1, k, dbase, ni)

    wci = _xform(dec_params["conv_in"]["w"])
    wci_i = None
    for c in range(6):
        if wci_i is None:
            srcs.append(wci)
            wci_i = len(srcs) - 1
        steps.append([(wci_i, 3, 0, 4, c)])
    for u in dec_params["ups"]:
        for rb in u["res"]:
            steps.append([add(_xform(rb["c1"]["w"]), 3, 0, 4),
                          add(_xform(rb["c2"]["w"]), 1, 3, 4)])
        steps.append([add(_xform(u["conv"]["w"]), 3, 0, 4)])
    steps.append([add(_xform(dec_params["conv_mid"]["w"]), 3, 0, 4)])
    co = jnp.pad(dec_params["conv_out"]["w"], ((0, _W - 263), (0, 0), (0, 0)))
    steps.append([add(_xform(co), 3, 0, 4)])
    return srcs, steps


def _make_dec_body(steps, nsrc):
    def body(*refs):
        wsrc = refs[:nsrc]
        zq_ref, b_ref, out_ref, act_ref, wbuf, *sems = refs[nsrc:]

        def copies(s):
            slot = s % _RING
            out = []
            for e in steps[s]:
                (si, k, dbase, ni), chunk = e[:4], (e[4] if len(e) > 4 else None)
                for j in range(k):
                    src = wsrc[si].at[j]
                    if chunk is not None:
                        src = wsrc[si].at[j, :, pl.ds(4 * chunk, 4)]
                    out.append(pltpu.make_async_copy(
                        src, wbuf.at[slot, dbase + j, :, :ni], sems[slot]))
            return out

        def wtap(l):
            return lambda j: _wmat(wbuf, l % _RING, j, 4)

        def cin(c):
            y = _conv3(zq_ref[c], wtap(c), None, 1)
            if c == 0:
                act_ref[:, :8, :] = y
            else:
                acc = act_ref[:, :8, :] + y
                if c == 5:
                    acc = jnp.maximum(acc + b_ref[c, 0][None, None, :], 0.0)
                act_ref[:, :8, :] = acc

        def res(l, t, dil):
            h = act_ref[:, :t, :]
            r = jnp.maximum(h, 0.0)
            r = _conv3(r, wtap(l), b_ref[l, 0], dil)
            r = jnp.maximum(r, 0.0)
            r = (lax.dot_general(r.reshape(_B * t, _W), _wmat(wbuf, l % _RING, 3, 4),
                                 (((1,), (1,)), ((), ())),
                                 preferred_element_type=jnp.float32)
                 + b_ref[l, 1][None, :]).reshape(_B, t, _W)
            act_ref[:, :t, :] = h + r

        def up(l, t):
            h = act_ref[:, :t, :]
            hr = jnp.broadcast_to(h[:, :, None, :], (_B, t, 2, _W))
            hr = hr.reshape(_B, 2 * t, _W)
            act_ref[:, :2 * t, :] = _conv3(hr, wtap(l), b_ref[l, 0], 1)

        for s0 in range(min(_RING - 1, len(steps))):
            for c in copies(s0):
                c.start()
        for s in range(len(steps)):
            if s + _RING - 1 < len(steps):
                for c in copies(s + _RING - 1):
                    c.start()
            for c in copies(s):
                c.wait()
            if s < 6:
                cin(s)
            elif s == 18:
                act_ref[:, :, :] = jnp.maximum(
                    _conv3(act_ref[:, :, :], wtap(s), b_ref[s, 0], 1), 0.0)
            elif s == 19:
                out_ref[:, :, :] = _conv3(act_ref[:, :, :], wtap(s),
                                          b_ref[s, 0], 1)
            else:
                blk = (s - 6) // 4
                r = (s - 6) % 4
                t = 8 << blk
                if r == 3:
                    up(s, t)
                else:
                    res(s, t, 3 ** (2 - r))

    return body


def _quant_body(feat_ref, emb_ref, idx_ref, loss_ref, perp_ref):
    z = feat_ref[0].reshape(_B * 8, _CODE_DIM)
    emb = emb_ref[0]
    prod = lax.dot_general(z, emb, (((1,), (1,)), ((), ())),
                           preferred_element_type=jnp.float32)
    d = (jnp.sum(z * z, axis=1, keepdims=True)
         + jnp.sum(emb * emb, axis=1)[None, :] - 2.0 * prod)
    idx = jnp.argmin(d, axis=1).astype(jnp.int32)
    onehot = (idx[:, None]
              == lax.broadcasted_iota(jnp.int32, (_B * 8, _NB_CODE), 1)
              ).astype(jnp.float32)
    zq = jnp.dot(onehot, emb, preferred_element_type=jnp.float32)
    loss = 2.0 * jnp.mean((zq - z) ** 2)
    e_mean = jnp.sum(onehot, axis=0) / float(_B * 8)
    perp = jnp.exp(-jnp.sum(e_mean * jnp.log(e_mean + 1e-10)))
    idx_ref[0, 0] = idx
    loss_ref[0, 0] = jnp.broadcast_to(loss, (128,))
    perp_ref[0, 0] = jnp.broadcast_to(perp, (128,))


def _sc_gather(emb_flat, gidx):
    info = plsc.get_sparse_core_info()
    nc, ns = info.num_cores, info.num_subcores
    bpw = 512 // (nc * ns)
    mesh = plsc.VectorSubcoreMesh(core_axis_name="c", subcore_axis_name="s")

    @functools.partial(
        pl.kernel, mesh=mesh,
        out_type=jax.ShapeDtypeStruct((512, _CODE_DIM), jnp.float32),
        scratch_types=[pltpu.VMEM((bpw,), jnp.int32),
                       pltpu.VMEM((bpw, _CODE_DIM), jnp.float32),
                       pltpu.SemaphoreType.DMA])
    def k(emb_hbm, idx_hbm, out_hbm, idx_v, rows_v, sem):
        wid = lax.axis_index("s") * nc + lax.axis_index("c")
        base = wid * bpw
        pltpu.sync_copy(idx_hbm.at[pl.ds(base, bpw)], idx_v)
        pltpu.async_copy(emb_hbm.at[idx_v], rows_v, sem).wait()
        pltpu.sync_copy(rows_v, out_hbm.at[pl.ds(base, bpw)])

    return k(emb_flat, gidx)


def _zb():
    return jnp.zeros((_W,), jnp.float32)


def kernel(x, enc_params, quant_emb, dec_params):
    parts = _part_indices()

    xps = []
    for idxs in parts:
        xc = jnp.transpose(jnp.take(x, jnp.array(idxs), axis=1), (0, 2, 1))
        xps.append(jnp.pad(xc, ((0, 0), (0, 0), (0, _CIN - len(idxs)))))
    x_parts = jnp.stack(xps)

    ebs = []
    for p in enc_params:
        bias2 = [jnp.stack([p["conv_in"]["b"], _zb()])]
        for dblk in p["downs"]:
            bias2.append(jnp.stack([dblk["down"]["b"], _zb()]))
            for rb in dblk["res"]:
                bias2.append(jnp.stack([rb["c1"]["b"], rb["c2"]["b"]]))
        bias2.append(jnp.stack([p["conv_out"]["b"], _zb()]))
        ebs.append(jnp.stack(bias2))
    enc_b = jnp.stack(ebs)

    dbs = []
    for c in range(6):
        dbs.append(jnp.stack(
            [dec_params["conv_in"]["b"] if c == 5 else _zb(), _zb()]))
    for u in dec_params["ups"]:
        for rb in u["res"]:
            dbs.append(jnp.stack([rb["c1"]["b"], rb["c2"]["b"]]))
        dbs.append(jnp.stack([u["conv"]["b"], _zb()]))
    dbs.append(jnp.stack([dec_params["conv_mid"]["b"], _zb()]))
    dbs.append(jnp.stack([jnp.pad(dec_params["conv_out"]["b"], (0, _W - 263)),
                          _zb()]))
    dec_b = jnp.stack(dbs)

    emb_stack = jnp.stack(quant_emb)

    esrcs, esteps = _enc_sources(enc_params)
    feat = pl.pallas_call(
        _make_enc_body(esteps, len(esrcs)),
        in_specs=[pl.BlockSpec(memory_space=pl.ANY)] * len(esrcs)
        + [pl.BlockSpec(memory_space=pltpu.VMEM)] * 2,
        out_shape=jax.ShapeDtypeStruct((_NPARTS, _B, 8, _W), jnp.float32),
        scratch_shapes=[pltpu.VMEM((_B, _T, _W), jnp.float32),
                        pltpu.VMEM((_RING, 4, 64, 4, 8, 128), jnp.float32)]
        + [pltpu.SemaphoreType.DMA] * _RING,
    )(*esrcs, x_parts, enc_b)

    idx, loss_arr, perp_arr = pl.pallas_call(
        _quant_body,
        grid=(_NPARTS,),
        in_specs=[
            pl.BlockSpec((1, _B, 8, _W), lambda p: (p, 0, 0, 0)),
            pl.BlockSpec((1, _NB_CODE, _CODE_DIM), lambda p: (p, 0, 0)),
        ],
        out_specs=[
            pl.BlockSpec((1, 1, 64), lambda p: (p, 0, 0)),
            pl.BlockSpec((1, 1, 128), lambda p: (p, 0, 0)),
            pl.BlockSpec((1, 1, 128), lambda p: (p, 0, 0)),
        ],
        out_shape=[
            jax.ShapeDtypeStruct((_NPARTS, 1, 64), jnp.int32),
            jax.ShapeDtypeStruct((_NPARTS, 1, 128), jnp.float32),
            jax.ShapeDtypeStruct((_NPARTS, 1, 128), jnp.float32),
        ],
        compiler_params=pltpu.CompilerParams(
            dimension_semantics=("arbitrary",)),
    )(feat, emb_stack)

    gidx = (idx.reshape(_NPARTS, 64)
            + _NB_CODE * jnp.arange(_NPARTS, dtype=jnp.int32)[:, None]
            ).reshape(-1)
    gidx = jnp.concatenate([gidx, jnp.zeros((128,), jnp.int32)])
    zq_rows = _sc_gather(emb_stack.reshape(-1, _CODE_DIM), gidx)
    zq = zq_rows[:_NPARTS * 64].reshape(_NPARTS, _B, 8, _CODE_DIM)

    dsrcs, dsteps = _dec_sources(dec_params)
    dec_out = pl.pallas_call(
        _make_dec_body(dsteps, len(dsrcs)),
        in_specs=[pl.BlockSpec(memory_space=pl.ANY)] * len(dsrcs)
        + [pl.BlockSpec(memory_space=pltpu.VMEM)] * 2,
        out_shape=jax.ShapeDtypeStruct((_B, _T, _W), jnp.float32),
        scratch_shapes=[pltpu.VMEM((_B, _T, _W), jnp.float32),
                        pltpu.VMEM((_RING, 4, 64, 4, 8, 128), jnp.float32)]
        + [pltpu.SemaphoreType.DMA] * _RING,
    )(*dsrcs, zq, dec_b)

    dec = jnp.transpose(dec_out[:, :, :263], (0, 2, 1))[:, :, None, :]
    loss = jnp.sum(loss_arr[:, 0, 0])
    perplexity = jnp.sum(perp_arr[:, 0, 0])
    return dec, loss, perplexity
